# baseline (speedup 1.0000x reference)
import jax
import jax.numpy as jnp
from jax import lax
from jax.experimental import pallas as pl
from jax.experimental.pallas import tpu as pltpu
from jax.experimental.pallas import tpu_sc as plsc

NN = 100000
NE = 1600000
CODES = 8450
NB = CODES * CODES
MEM_SHAPE = (65, 65, 2, 65, 65, 2)
TSHAPE = (65, 65, 2, 65, 2, 65)
PERM = (0, 1, 2, 3, 5, 4)

NC, NS = 2, 16

EPW = NE // (NC * NS)
ECH = 2000
NECH = EPW // ECH
ACH = 2000
NACH = NN // ACH
FPW = 102400
NF = NC * NS * FPW

BCH = 8192
NBCH = 13
TSL = BCH * NBCH
W = TSL * NS
NPASS = -(-NB // (2 * W))
HSZ = W + 64
SCH = 2048
FPT = NF // NS
NSCH = FPT // SCH
TAIL = NB - (2 * NPASS - 1) * W - 14 * TSL - 6 * BCH

_mesh = plsc.VectorSubcoreMesh(core_axis_name="c", subcore_axis_name="s")
_params = pltpu.CompilerParams(needs_layout_passes=False)


def _quant(y):
    r = (y + 0.5).astype(jnp.int32)
    rf = r.astype(jnp.float32)
    tie = (rf - y) == 0.5
    odd = (r & 1) == 1
    r = r - jnp.where(tie & odd, 1, 0)
    return jnp.clip(r, 0, 64)


def _k1_body(xs, ys, tex, ea, eb, flats, pv, xsv, ysv, txv, eav, ebv, f1v, f2v):
    wid = lax.axis_index("s") * NC + lax.axis_index("c")

    def a_chunk(ci, _):
        base = ci * ACH
        pltpu.sync_copy(xs.at[pl.ds(base, ACH)], xsv)
        pltpu.sync_copy(ys.at[pl.ds(base, ACH)], ysv)
        pltpu.sync_copy(tex.at[pl.ds(base, ACH)], txv)

        def vec(i, _):
            o = i * 16
            qx = _quant(xsv[pl.ds(o, 16)] + 32.0)
            qy = _quant(ysv[pl.ds(o, 16)] + 32.0)
            t = txv[pl.ds(o, 16)]
            tq = jnp.where(t >= 0.5, 1, 0).astype(jnp.int32)
            tq = jnp.where(t == 0.5, 0, tq)
            lcode = (qx * 65 + qy) * 2 + tq
            rcode = (qx * 2 + tq) * 65 + qy
            pv[pl.ds(base + o, 16)] = lcode * 16384 + rcode
            return 0

        lax.fori_loop(0, ACH // 16, vec, 0)
        return 0

    lax.fori_loop(0, NACH, a_chunk, 0)

    ebase = wid * EPW
    fbase = wid * FPW

    def b_chunk(ci, _):
        off = ci * ECH
        pltpu.sync_copy(ea.at[pl.ds(ebase + off, ECH)], eav)
        pltpu.sync_copy(eb.at[pl.ds(ebase + off, ECH)], ebv)

        def vec(i, _):
            o = i * 16
            pa = plsc.load_gather(pv, [eav[pl.ds(o, 16)]])
            pb = plsc.load_gather(pv, [ebv[pl.ds(o, 16)]])
            f1v[pl.ds(o, 16)] = (pa >> 14) * CODES + (pb & 16383)
            f2v[pl.ds(o, 16)] = (pb >> 14) * CODES + (pa & 16383)
            return 0

        lax.fori_loop(0, ECH // 16, vec, 0)
        pltpu.sync_copy(f1v, flats.at[pl.ds(fbase + off, ECH)])
        pltpu.sync_copy(f2v, flats.at[pl.ds(fbase + EPW + off, ECH)])
        return 0

    lax.fori_loop(0, NECH, b_chunk, 0)

    neg1 = jnp.full((16,), -1, jnp.int32)

    def fill(i, _):
        f1v[pl.ds(i * 16, 16)] = neg1
        return 0

    lax.fori_loop(0, ECH // 16, fill, 0)
    pltpu.sync_copy(f1v, flats.at[pl.ds(fbase + 2 * EPW, ECH)])
    pltpu.sync_copy(f1v.at[pl.ds(0, 400)],
                    flats.at[pl.ds(fbase + 2 * EPW + ECH, 400)])


def _k2_body(flats, memf, outf, fv0, fv1, onesv, bnc0, bnc1, hist,
             dsem, ssem, msa, msb):
    core = lax.axis_index("c")
    sid = lax.axis_index("s")
    lane = lax.broadcasted_iota(jnp.int32, (16,), 0)

    def ones_init(i, _):
        onesv[pl.ds(i * 16, 16)] = jnp.full((16,), 1.0, jnp.float32)
        return 0

    lax.fori_loop(0, SCH // 16, ones_init, 0)

    bn = (bnc0, bnc1)

    def do_pass(p, _):
        wb = (p * 2 + core) * W

        def hbm_sl(ref, j, n=BCH):
            return ref.at[pl.ds(wb + sid * TSL + j * BCH, n)]

        def sp_sl(j, n=BCH):
            return hist.at[pl.ds(sid * TSL + j * BCH, n)]

        def move_async(to_hbm):
            h1 = [None, None]
            h2 = [None, None]

            def fire1(j, x):
                if to_hbm:
                    h1[x] = pltpu.async_copy(sp_sl(j), bn[x], msa)
                else:
                    h1[x] = pltpu.async_copy(hbm_sl(memf, j), bn[x], msa)

            def fire2(j, x):
                if to_hbm:
                    h2[x] = pltpu.async_copy(bn[x], hbm_sl(outf, j), msb)
                else:
                    h2[x] = pltpu.async_copy(bn[x], sp_sl(j), msb)

            for j in range(NBCH):
                x = j & 1
                if h2[x] is not None:
                    h2[x].wait()
                fire1(j, x)
                y = x ^ 1
                if h1[y] is not None:
                    h1[y].wait()
                    fire2(j - 1, y)
            x = (NBCH - 1) & 1
            h1[x].wait()
            fire2(NBCH - 1, x)
            h2[x ^ 1].wait()
            h2[x].wait()

        def move_sync(j, to_hbm):
            start = wb + sid * TSL + j * BCH
            full = start + BCH <= NB
            part = jnp.logical_and(start < NB, jnp.logical_not(full))

            @pl.when(full)
            def _():
                if to_hbm:
                    pltpu.sync_copy(sp_sl(j), bnc0)
                    pltpu.sync_copy(bnc0, hbm_sl(outf, j))
                else:
                    pltpu.sync_copy(hbm_sl(memf, j), bnc0)
                    pltpu.sync_copy(bnc0, sp_sl(j))

            @pl.when(part)
            def _():
                if to_hbm:
                    pltpu.sync_copy(sp_sl(j, TAIL), bnc0.at[pl.ds(0, TAIL)])
                    pltpu.sync_copy(bnc0.at[pl.ds(0, TAIL)],
                                    hbm_sl(outf, j, TAIL))
                else:
                    pltpu.sync_copy(hbm_sl(memf, j, TAIL),
                                    bnc0.at[pl.ds(0, TAIL)])
                    pltpu.sync_copy(bnc0.at[pl.ds(0, TAIL)], sp_sl(j, TAIL))

        @pl.when(p < NPASS - 1)
        def _():
            move_async(False)

        @pl.when(p == NPASS - 1)
        def _():
            def load_j(j, _):
                move_sync(j, False)
                return 0

            lax.fori_loop(0, NBCH, load_j, 0)

        plsc.subcore_barrier()

        @pl.when(wb < NB)
        def _():
            base = sid * FPT
            bufs = (fv0, fv1)

            def scan_buf(buf):
                def vec(i, _):
                    for u8 in range(8):
                        o = (i * 8 + u8) * 16
                        v = buf[pl.ds(o, 16)]
                        rel = v - wb
                        ok = (rel >= 0) & (rel < W)
                        buf[pl.ds(o, 16)] = jnp.where(ok, rel, W + lane)
                    return 0

                lax.fori_loop(0, SCH // 128, vec, 0)

            def dma_start(ci, buf):
                pltpu.async_copy(flats.at[pl.ds(base + ci * SCH, SCH)], buf,
                                 dsem)

            def dma_drain(ci, buf):
                pltpu.make_async_copy(flats.at[pl.ds(base + ci * SCH, SCH)],
                                      buf, dsem).wait()

            def sc_fire(buf):
                pltpu.async_copy(onesv, hist.at[buf], ssem, add=True)

            def sc_drain(buf):
                pltpu.make_async_copy(onesv, hist.at[buf], ssem).wait()

            dma_start(0, fv0)
            dma_start(1, fv1)

            def super_chunk(g, _):
                for u in range(2):
                    buf = bufs[u]
                    ci = g * 2 + u
                    dma_drain(ci, buf)
                    scan_buf(buf)
                    sc_fire(buf)
                    sc_drain(buf)

                    @pl.when(g < (NSCH // 2) - 1)
                    def _():
                        dma_start(ci + 2, buf)

                return 0

            lax.fori_loop(0, NSCH // 2, super_chunk, 0)

        plsc.subcore_barrier()

        @pl.when(p < NPASS - 1)
        def _():
            move_async(True)

        @pl.when(p == NPASS - 1)
        def _():
            def store_j(j, _):
                move_sync(j, True)
                return 0

            lax.fori_loop(0, NBCH, store_j, 0)

        return 0

    lax.fori_loop(0, NPASS, do_pass, 0)


_k1 = pl.kernel(
    _k1_body,
    out_type=jax.ShapeDtypeStruct((NF,), jnp.int32),
    mesh=_mesh,
    compiler_params=_params,
    scratch_types=[
        pltpu.VMEM((NN,), jnp.int32),
        pltpu.VMEM((ACH,), jnp.float32),
        pltpu.VMEM((ACH,), jnp.float32),
        pltpu.VMEM((ACH,), jnp.float32),
        pltpu.VMEM((ECH,), jnp.int32),
        pltpu.VMEM((ECH,), jnp.int32),
        pltpu.VMEM((ECH,), jnp.int32),
        pltpu.VMEM((ECH,), jnp.int32),
    ],
)

_k2 = pl.kernel(
    _k2_body,
    out_type=jax.ShapeDtypeStruct((NB,), jnp.float32),
    mesh=_mesh,
    compiler_params=_params,
    scratch_types=[
        pltpu.VMEM((SCH,), jnp.int32),
        pltpu.VMEM((SCH,), jnp.int32),
        pltpu.VMEM((SCH,), jnp.float32),
        pltpu.VMEM((BCH,), jnp.float32),
        pltpu.VMEM((BCH,), jnp.float32),
        pltpu.VMEM_SHARED((HSZ,), jnp.float32),
        pltpu.SemaphoreType.DMA,
        pltpu.SemaphoreType.DMA,
        pltpu.SemaphoreType.DMA,
        pltpu.SemaphoreType.DMA,
    ],
)


@jax.jit
def kernel(pts, tex, edges, mem):
    xs = pts[:, 0]
    ys = pts[:, 1]
    ea = edges[:, 0]
    eb = edges[:, 1]
    memf = mem.transpose(PERM).reshape(-1)
    flats = _k1(xs, ys, tex, ea, eb)
    outp = _k2(flats, memf)
    return outp.reshape(TSHAPE).transpose(PERM)


# speedup vs baseline: 1.2940x; 1.2940x over previous
import jax
import jax.numpy as jnp
from jax import lax
from jax.experimental import pallas as pl
from jax.experimental.pallas import tpu as pltpu
from jax.experimental.pallas import tpu_sc as plsc

NN = 100000
NE = 1600000
CODES = 8450
NB = CODES * CODES
MEM_SHAPE = (65, 65, 2, 65, 65, 2)
TSHAPE = (65, 65, 2, 65, 2, 65)
PERM = (0, 1, 2, 3, 5, 4)

NC, NS = 2, 16

EPW = NE // (NC * NS)
ECH = 2000
NECH = EPW // ECH
ACH = 2000
NACH = NN // ACH
FPW = 102400
NF = NC * NS * FPW

BCH = 6656
NBCH = 16
TSL = BCH * NBCH
W = TSL * NS
NPASS = -(-NB // (2 * W))
HSZ = W + 64
SCH = 2048
FPT = NF // NS
NSCH = FPT // SCH
SBN = 2048
TAIL = NB - (2 * NPASS - 1) * W - 14 * TSL - 7 * BCH

_mesh = plsc.VectorSubcoreMesh(core_axis_name="c", subcore_axis_name="s")
_params = pltpu.CompilerParams(needs_layout_passes=False)


def _quant(y):
    r = (y + 0.5).astype(jnp.int32)
    rf = r.astype(jnp.float32)
    tie = (rf - y) == 0.5
    odd = (r & 1) == 1
    r = r - jnp.where(tie & odd, 1, 0)
    return jnp.clip(r, 0, 64)


def _k1_body(xs, ys, tex, ea, eb, flats, pv, xsv, ysv, txv, eav, ebv, f1v, f2v):
    wid = lax.axis_index("s") * NC + lax.axis_index("c")

    def a_chunk(ci, _):
        base = ci * ACH
        pltpu.sync_copy(xs.at[pl.ds(base, ACH)], xsv)
        pltpu.sync_copy(ys.at[pl.ds(base, ACH)], ysv)
        pltpu.sync_copy(tex.at[pl.ds(base, ACH)], txv)

        def vec(i, _):
            o = i * 16
            qx = _quant(xsv[pl.ds(o, 16)] + 32.0)
            qy = _quant(ysv[pl.ds(o, 16)] + 32.0)
            t = txv[pl.ds(o, 16)]
            tq = jnp.where(t >= 0.5, 1, 0).astype(jnp.int32)
            tq = jnp.where(t == 0.5, 0, tq)
            lcode = (qx * 65 + qy) * 2 + tq
            rcode = (qx * 2 + tq) * 65 + qy
            pv[pl.ds(base + o, 16)] = lcode * 16384 + rcode
            return 0

        lax.fori_loop(0, ACH // 16, vec, 0)
        return 0

    lax.fori_loop(0, NACH, a_chunk, 0)

    ebase = wid * EPW
    fbase = wid * FPW

    def b_chunk(ci, _):
        off = ci * ECH
        pltpu.sync_copy(ea.at[pl.ds(ebase + off, ECH)], eav)
        pltpu.sync_copy(eb.at[pl.ds(ebase + off, ECH)], ebv)

        def vec(i, _):
            o = i * 16
            pa = plsc.load_gather(pv, [eav[pl.ds(o, 16)]])
            pb = plsc.load_gather(pv, [ebv[pl.ds(o, 16)]])
            f1v[pl.ds(o, 16)] = (pa >> 14) * CODES + (pb & 16383)
            f2v[pl.ds(o, 16)] = (pb >> 14) * CODES + (pa & 16383)
            return 0

        lax.fori_loop(0, ECH // 16, vec, 0)
        pltpu.sync_copy(f1v, flats.at[pl.ds(fbase + off, ECH)])
        pltpu.sync_copy(f2v, flats.at[pl.ds(fbase + EPW + off, ECH)])
        return 0

    lax.fori_loop(0, NECH, b_chunk, 0)

    neg1 = jnp.full((16,), -1, jnp.int32)

    def fill(i, _):
        f1v[pl.ds(i * 16, 16)] = neg1
        return 0

    lax.fori_loop(0, ECH // 16, fill, 0)
    pltpu.sync_copy(f1v, flats.at[pl.ds(fbase + 2 * EPW, ECH)])
    pltpu.sync_copy(f1v.at[pl.ds(0, 400)],
                    flats.at[pl.ds(fbase + 2 * EPW + ECH, 400)])


def _k2_body(flats, memf, outf, fv0, fv1, onesv, stg, sbuf, bnc0, bnc1, hist,
             dsem, ssem, msa, msb):
    core = lax.axis_index("c")
    sid = lax.axis_index("s")
    lane = lax.broadcasted_iota(jnp.int32, (16,), 0)

    def ones_init(i, _):
        onesv[pl.ds(i * 16, 16)] = jnp.full((16,), 1.0, jnp.float32)
        return 0

    lax.fori_loop(0, SCH // 16, ones_init, 0)

    bn = (bnc0, bnc1)

    def do_pass(p, _):
        wb = (p * 2 + core) * W

        def hbm_sl(ref, j, n=BCH):
            return ref.at[pl.ds(wb + sid * TSL + j * BCH, n)]

        def sp_sl(j, n=BCH):
            return hist.at[pl.ds(sid * TSL + j * BCH, n)]

        def move_async(to_hbm):
            h1 = [None, None]
            h2 = [None, None]

            def fire1(j, x):
                if to_hbm:
                    h1[x] = pltpu.async_copy(sp_sl(j), bn[x], msa)
                else:
                    h1[x] = pltpu.async_copy(hbm_sl(memf, j), bn[x], msa)

            def fire2(j, x):
                if to_hbm:
                    h2[x] = pltpu.async_copy(bn[x], hbm_sl(outf, j), msb)
                else:
                    h2[x] = pltpu.async_copy(bn[x], sp_sl(j), msb)

            for j in range(NBCH):
                x = j & 1
                if h2[x] is not None:
                    h2[x].wait()
                fire1(j, x)
                y = x ^ 1
                if h1[y] is not None:
                    h1[y].wait()
                    fire2(j - 1, y)
            x = (NBCH - 1) & 1
            h1[x].wait()
            fire2(NBCH - 1, x)
            h2[x ^ 1].wait()
            h2[x].wait()

        def move_sync(j, to_hbm):
            start = wb + sid * TSL + j * BCH
            full = start + BCH <= NB
            part = jnp.logical_and(start < NB, jnp.logical_not(full))

            @pl.when(full)
            def _():
                if to_hbm:
                    pltpu.sync_copy(sp_sl(j), bnc0)
                    pltpu.sync_copy(bnc0, hbm_sl(outf, j))
                else:
                    pltpu.sync_copy(hbm_sl(memf, j), bnc0)
                    pltpu.sync_copy(bnc0, sp_sl(j))

            @pl.when(part)
            def _():
                if to_hbm:
                    pltpu.sync_copy(sp_sl(j, TAIL), bnc0.at[pl.ds(0, TAIL)])
                    pltpu.sync_copy(bnc0.at[pl.ds(0, TAIL)],
                                    hbm_sl(outf, j, TAIL))
                else:
                    pltpu.sync_copy(hbm_sl(memf, j, TAIL),
                                    bnc0.at[pl.ds(0, TAIL)])
                    pltpu.sync_copy(bnc0.at[pl.ds(0, TAIL)], sp_sl(j, TAIL))

        @pl.when(p < NPASS - 1)
        def _():
            move_async(False)

        @pl.when(p == NPASS - 1)
        def _():
            def load_j(j, _):
                move_sync(j, False)
                return 0

            lax.fori_loop(0, NBCH, load_j, 0)

        plsc.subcore_barrier()

        @pl.when(wb < NB)
        def _():
            base = sid * FPT
            bufs = (fv0, fv1)

            def flush():
                def cp(i, _):
                    o = i * 16
                    sbuf[pl.ds(o, 16)] = stg[pl.ds(o, 16)]
                    return 0

                lax.fori_loop(0, SBN // 16, cp, 0)
                pltpu.async_copy(onesv, hist.at[sbuf], ssem, add=True).wait()

            def scan_buf(buf, cnt):
                def vec(i, cnt):
                    v = buf[pl.ds(i * 16, 16)]
                    rel = v - wb
                    ok = (rel >= 0) & (rel < W)
                    plsc.store_compressed(stg.at[pl.ds(cnt, 16)], rel, mask=ok)
                    cnt = cnt + plsc.all_reduce_population_count(ok)[0]
                    do_flush = cnt >= SBN

                    @pl.when(do_flush)
                    def _():
                        flush()
                        left = stg[pl.ds(SBN, 16)]
                        plsc.store_scatter(stg, [lane], left,
                                           mask=lane < (cnt - SBN))

                    return jnp.where(do_flush, cnt - SBN, cnt)

                return lax.fori_loop(0, SCH // 16, vec, cnt)

            def dma_start(ci, buf):
                pltpu.async_copy(flats.at[pl.ds(base + ci * SCH, SCH)], buf,
                                 dsem)

            def dma_drain(ci, buf):
                pltpu.make_async_copy(flats.at[pl.ds(base + ci * SCH, SCH)],
                                      buf, dsem).wait()

            dma_start(0, fv0)
            dma_start(1, fv1)

            def super_chunk(g, cnt):
                for u in range(2):
                    buf = bufs[u]
                    ci = g * 2 + u
                    dma_drain(ci, buf)
                    cnt = scan_buf(buf, cnt)

                    @pl.when(g < (NSCH // 2) - 1)
                    def _():
                        dma_start(ci + 2, buf)

                return cnt

            cnt = lax.fori_loop(0, NSCH // 2, super_chunk, 0)
            dumpvec = W + lane
            nk = (SBN - cnt + 15) >> 4

            def padv(k, _):
                stg[pl.ds(cnt + k * 16, 16)] = dumpvec
                return 0

            lax.fori_loop(0, nk, padv, 0)
            flush()

        plsc.subcore_barrier()

        @pl.when(p < NPASS - 1)
        def _():
            move_async(True)

        @pl.when(p == NPASS - 1)
        def _():
            def store_j(j, _):
                move_sync(j, True)
                return 0

            lax.fori_loop(0, NBCH, store_j, 0)

        return 0

    lax.fori_loop(0, NPASS, do_pass, 0)


_k1 = pl.kernel(
    _k1_body,
    out_type=jax.ShapeDtypeStruct((NF,), jnp.int32),
    mesh=_mesh,
    compiler_params=_params,
    scratch_types=[
        pltpu.VMEM((NN,), jnp.int32),
        pltpu.VMEM((ACH,), jnp.float32),
        pltpu.VMEM((ACH,), jnp.float32),
        pltpu.VMEM((ACH,), jnp.float32),
        pltpu.VMEM((ECH,), jnp.int32),
        pltpu.VMEM((ECH,), jnp.int32),
        pltpu.VMEM((ECH,), jnp.int32),
        pltpu.VMEM((ECH,), jnp.int32),
    ],
)

_k2 = pl.kernel(
    _k2_body,
    out_type=jax.ShapeDtypeStruct((NB,), jnp.float32),
    mesh=_mesh,
    compiler_params=_params,
    scratch_types=[
        pltpu.VMEM((SCH,), jnp.int32),
        pltpu.VMEM((SCH,), jnp.int32),
        pltpu.VMEM((SBN,), jnp.float32),
        pltpu.VMEM((SBN + 16,), jnp.int32),
        pltpu.VMEM((SBN,), jnp.int32),
        pltpu.VMEM((BCH,), jnp.float32),
        pltpu.VMEM((BCH,), jnp.float32),
        pltpu.VMEM_SHARED((HSZ,), jnp.float32),
        pltpu.SemaphoreType.DMA,
        pltpu.SemaphoreType.DMA,
        pltpu.SemaphoreType.DMA,
        pltpu.SemaphoreType.DMA,
    ],
)


@jax.jit
def kernel(pts, tex, edges, mem):
    xs = pts[:, 0]
    ys = pts[:, 1]
    ea = edges[:, 0]
    eb = edges[:, 1]
    memf = mem.transpose(PERM).reshape(-1)
    flats = _k1(xs, ys, tex, ea, eb)
    outp = _k2(flats, memf)
    return outp.reshape(TSHAPE).transpose(PERM)


# speedup vs baseline: 1.7789x; 1.3747x over previous
import jax
import jax.numpy as jnp
from jax import lax
from jax.experimental import pallas as pl
from jax.experimental.pallas import tpu as pltpu
from jax.experimental.pallas import tpu_sc as plsc

NN = 100000
NE = 1600000
CODES = 8450
NB = CODES * CODES
MEM_SHAPE = (65, 65, 2, 65, 65, 2)
TSHAPE = (65, 65, 2, 65, 2, 65)
PERM = (0, 1, 2, 3, 5, 4)

NC, NS = 2, 16

EPW = NE // (NC * NS)
ECH = 2000
NECH = EPW // ECH
ACH = 2000
NACH = NN // ACH
FPW = 102400
NF = NC * NS * FPW

BCH = 6656
NBCH = 16
TSL = BCH * NBCH
W = TSL * NS
NPASS = -(-NB // (2 * W))
HSZ = W + 64
SCH = 2048
FPT = NF // NS
NSCH = FPT // SCH
SBN = 2048
TAIL = NB - (2 * NPASS - 1) * W - 14 * TSL - 7 * BCH

_mesh = plsc.VectorSubcoreMesh(core_axis_name="c", subcore_axis_name="s")
_params = pltpu.CompilerParams(needs_layout_passes=False)


def _quant(y):
    r = (y + 0.5).astype(jnp.int32)
    rf = r.astype(jnp.float32)
    tie = (rf - y) == 0.5
    odd = (r & 1) == 1
    r = r - jnp.where(tie & odd, 1, 0)
    return jnp.clip(r, 0, 64)


def _k1_body(xs, ys, tex, ea, eb, flats, pv, xsv, ysv, txv, eav, ebv, f1v, f2v):
    wid = lax.axis_index("s") * NC + lax.axis_index("c")

    def a_chunk(ci, _):
        base = ci * ACH
        pltpu.sync_copy(xs.at[pl.ds(base, ACH)], xsv)
        pltpu.sync_copy(ys.at[pl.ds(base, ACH)], ysv)
        pltpu.sync_copy(tex.at[pl.ds(base, ACH)], txv)

        def vec(i, _):
            o = i * 16
            qx = _quant(xsv[pl.ds(o, 16)] + 32.0)
            qy = _quant(ysv[pl.ds(o, 16)] + 32.0)
            t = txv[pl.ds(o, 16)]
            tq = jnp.where(t >= 0.5, 1, 0).astype(jnp.int32)
            tq = jnp.where(t == 0.5, 0, tq)
            lcode = (qx * 65 + qy) * 2 + tq
            rcode = (qx * 2 + tq) * 65 + qy
            pv[pl.ds(base + o, 16)] = lcode * 16384 + rcode
            return 0

        lax.fori_loop(0, ACH // 16, vec, 0)
        return 0

    lax.fori_loop(0, NACH, a_chunk, 0)

    ebase = wid * EPW
    fbase = wid * FPW

    def b_chunk(ci, _):
        off = ci * ECH
        pltpu.sync_copy(ea.at[pl.ds(ebase + off, ECH)], eav)
        pltpu.sync_copy(eb.at[pl.ds(ebase + off, ECH)], ebv)

        def vec(i, _):
            o = i * 16
            pa = plsc.load_gather(pv, [eav[pl.ds(o, 16)]])
            pb = plsc.load_gather(pv, [ebv[pl.ds(o, 16)]])
            f1v[pl.ds(o, 16)] = (pa >> 14) * CODES + (pb & 16383)
            f2v[pl.ds(o, 16)] = (pb >> 14) * CODES + (pa & 16383)
            return 0

        lax.fori_loop(0, ECH // 16, vec, 0)
        pltpu.sync_copy(f1v, flats.at[pl.ds(fbase + off, ECH)])
        pltpu.sync_copy(f2v, flats.at[pl.ds(fbase + EPW + off, ECH)])
        return 0

    lax.fori_loop(0, NECH, b_chunk, 0)

    neg1 = jnp.full((16,), -1, jnp.int32)

    def fill(i, _):
        f1v[pl.ds(i * 16, 16)] = neg1
        return 0

    lax.fori_loop(0, ECH // 16, fill, 0)
    pltpu.sync_copy(f1v, flats.at[pl.ds(fbase + 2 * EPW, ECH)])
    pltpu.sync_copy(f1v.at[pl.ds(0, 400)],
                    flats.at[pl.ds(fbase + 2 * EPW + ECH, 400)])


def _k2_body(flats, memf, outf, fv0, fv1, onesv, stg, sbuf, bnc0, bnc1, hist,
             dsem, ssem, msa, msb):
    core = lax.axis_index("c")
    sid = lax.axis_index("s")
    lane = lax.broadcasted_iota(jnp.int32, (16,), 0)

    def ones_init(i, _):
        onesv[pl.ds(i * 16, 16)] = jnp.full((16,), 1.0, jnp.float32)
        return 0

    lax.fori_loop(0, SCH // 16, ones_init, 0)

    bn = (bnc0, bnc1)

    def do_pass(p, _):
        wb = (p * 2 + core) * W

        def hbm_sl(ref, j, n=BCH):
            return ref.at[pl.ds(wb + sid * TSL + j * BCH, n)]

        def sp_sl(j, n=BCH):
            return hist.at[pl.ds(sid * TSL + j * BCH, n)]

        def move_async(to_hbm):
            h1 = [None, None]
            h2 = [None, None]

            def fire1(j, x):
                if to_hbm:
                    h1[x] = pltpu.async_copy(sp_sl(j), bn[x], msa)
                else:
                    h1[x] = pltpu.async_copy(hbm_sl(memf, j), bn[x], msa)

            def fire2(j, x):
                if to_hbm:
                    h2[x] = pltpu.async_copy(bn[x], hbm_sl(outf, j), msb)
                else:
                    h2[x] = pltpu.async_copy(bn[x], sp_sl(j), msb)

            for j in range(NBCH):
                x = j & 1
                if h2[x] is not None:
                    h2[x].wait()
                fire1(j, x)
                y = x ^ 1
                if h1[y] is not None:
                    h1[y].wait()
                    fire2(j - 1, y)
            x = (NBCH - 1) & 1
            h1[x].wait()
            fire2(NBCH - 1, x)
            h2[x ^ 1].wait()
            h2[x].wait()

        def move_sync(j, to_hbm):
            start = wb + sid * TSL + j * BCH
            full = start + BCH <= NB
            part = jnp.logical_and(start < NB, jnp.logical_not(full))

            @pl.when(full)
            def _():
                if to_hbm:
                    pltpu.sync_copy(sp_sl(j), bnc0)
                    pltpu.sync_copy(bnc0, hbm_sl(outf, j))
                else:
                    pltpu.sync_copy(hbm_sl(memf, j), bnc0)
                    pltpu.sync_copy(bnc0, sp_sl(j))

            @pl.when(part)
            def _():
                if to_hbm:
                    pltpu.sync_copy(sp_sl(j, TAIL), bnc0.at[pl.ds(0, TAIL)])
                    pltpu.sync_copy(bnc0.at[pl.ds(0, TAIL)],
                                    hbm_sl(outf, j, TAIL))
                else:
                    pltpu.sync_copy(hbm_sl(memf, j, TAIL),
                                    bnc0.at[pl.ds(0, TAIL)])
                    pltpu.sync_copy(bnc0.at[pl.ds(0, TAIL)], sp_sl(j, TAIL))

        @pl.when(p < NPASS - 1)
        def _():
            move_async(False)

        @pl.when(p == NPASS - 1)
        def _():
            def load_j(j, _):
                move_sync(j, False)
                return 0

            lax.fori_loop(0, NBCH, load_j, 0)

        plsc.subcore_barrier()

        @pl.when(wb < NB)
        def _():
            base = sid * FPT
            bufs = (fv0, fv1)

            def flush():
                def cp(i, _):
                    o = i * 16
                    sbuf[pl.ds(o, 16)] = stg[pl.ds(o, 16)]
                    return 0

                lax.fori_loop(0, SBN // 16, cp, 0)
                pltpu.async_copy(onesv, hist.at[sbuf], ssem, add=True).wait()

            def scan_buf(buf, cnt):
                def vec(i, cnt):
                    for u8 in range(4):
                        o = (i * 4 + u8) * 16
                        v = buf[pl.ds(o, 16)]
                        rel = v - wb
                        ok = (rel >= 0) & (rel < W)
                        plsc.store_compressed(stg.at[pl.ds(cnt, 16)], rel,
                                              mask=ok)
                        cnt = cnt + plsc.all_reduce_population_count(ok)[0]
                    do_flush = cnt >= SBN

                    @pl.when(do_flush)
                    def _():
                        flush()
                        for k in range(4):
                            left = stg[pl.ds(SBN + k * 16, 16)]
                            plsc.store_scatter(
                                stg, [lane + k * 16], left,
                                mask=(lane + k * 16) < (cnt - SBN))

                    return jnp.where(do_flush, cnt - SBN, cnt)

                return lax.fori_loop(0, SCH // 64, vec, cnt)

            def dma_start(ci, buf):
                pltpu.async_copy(flats.at[pl.ds(base + ci * SCH, SCH)], buf,
                                 dsem)

            def dma_drain(ci, buf):
                pltpu.make_async_copy(flats.at[pl.ds(base + ci * SCH, SCH)],
                                      buf, dsem).wait()

            dma_start(0, fv0)
            dma_start(1, fv1)

            def super_chunk(g, cnt):
                for u in range(2):
                    buf = bufs[u]
                    ci = g * 2 + u
                    dma_drain(ci, buf)
                    cnt = scan_buf(buf, cnt)

                    @pl.when(g < (NSCH // 2) - 1)
                    def _():
                        dma_start(ci + 2, buf)

                return cnt

            cnt = lax.fori_loop(0, NSCH // 2, super_chunk, 0)
            dumpvec = W + lane
            nk = (SBN - cnt + 15) >> 4

            def padv(k, _):
                stg[pl.ds(cnt + k * 16, 16)] = dumpvec
                return 0

            lax.fori_loop(0, nk, padv, 0)
            flush()

        plsc.subcore_barrier()

        @pl.when(p < NPASS - 1)
        def _():
            move_async(True)

        @pl.when(p == NPASS - 1)
        def _():
            def store_j(j, _):
                move_sync(j, True)
                return 0

            lax.fori_loop(0, NBCH, store_j, 0)

        return 0

    lax.fori_loop(0, NPASS, do_pass, 0)


_k1 = pl.kernel(
    _k1_body,
    out_type=jax.ShapeDtypeStruct((NF,), jnp.int32),
    mesh=_mesh,
    compiler_params=_params,
    scratch_types=[
        pltpu.VMEM((NN,), jnp.int32),
        pltpu.VMEM((ACH,), jnp.float32),
        pltpu.VMEM((ACH,), jnp.float32),
        pltpu.VMEM((ACH,), jnp.float32),
        pltpu.VMEM((ECH,), jnp.int32),
        pltpu.VMEM((ECH,), jnp.int32),
        pltpu.VMEM((ECH,), jnp.int32),
        pltpu.VMEM((ECH,), jnp.int32),
    ],
)

_k2 = pl.kernel(
    _k2_body,
    out_type=jax.ShapeDtypeStruct((NB,), jnp.float32),
    mesh=_mesh,
    compiler_params=_params,
    scratch_types=[
        pltpu.VMEM((SCH,), jnp.int32),
        pltpu.VMEM((SCH,), jnp.int32),
        pltpu.VMEM((SBN,), jnp.float32),
        pltpu.VMEM((SBN + 80,), jnp.int32),
        pltpu.VMEM((SBN,), jnp.int32),
        pltpu.VMEM((BCH,), jnp.float32),
        pltpu.VMEM((BCH,), jnp.float32),
        pltpu.VMEM_SHARED((HSZ,), jnp.float32),
        pltpu.SemaphoreType.DMA,
        pltpu.SemaphoreType.DMA,
        pltpu.SemaphoreType.DMA,
        pltpu.SemaphoreType.DMA,
    ],
)


@jax.jit
def kernel(pts, tex, edges, mem):
    xs = pts[:, 0]
    ys = pts[:, 1]
    ea = edges[:, 0]
    eb = edges[:, 1]
    memf = mem.transpose(PERM).reshape(-1)
    flats = _k1(xs, ys, tex, ea, eb)
    outp = _k2(flats, memf)
    return outp.reshape(TSHAPE).transpose(PERM)
